# baseline (speedup 1.0000x reference)
import functools

import jax
import jax.numpy as jnp
from jax.experimental import pallas as pl
from jax.experimental.pallas import tpu as pltpu

N = 10000
E = 320000
IN_DIM = 128
HID = 128
EPS = 0.3


def _lrelu(x):
    return jnp.where(x >= 0, x, 0.3 * x)


def _mm_lrelu_body(x_ref, w_ref, b_ref, o_ref):
    o_ref[...] = _lrelu(
        jnp.dot(x_ref[...], w_ref[...], preferred_element_type=jnp.float32)
        + b_ref[...]
    )


def _mm_lrelu(x, w, b, block=2000):
    n, k = x.shape
    m = w.shape[1]
    grid = (n // block,)
    return pl.pallas_call(
        _mm_lrelu_body,
        grid=grid,
        in_specs=[
            pl.BlockSpec((block, k), lambda i: (i, 0)),
            pl.BlockSpec((k, m), lambda i: (0, 0)),
            pl.BlockSpec((1, m), lambda i: (0, 0)),
        ],
        out_specs=pl.BlockSpec((block, m), lambda i: (i, 0)),
        out_shape=jax.ShapeDtypeStruct((n, m), jnp.float32),
    )(x, w, b.reshape(1, m))


def _deg_norm(dst):
    deg = jnp.zeros((N,), jnp.float32).at[dst].add(1.0)
    return jax.lax.rsqrt(deg + 1.0)


def _layer(hf, pd, ps, src, dst, d):
    a = jnp.tanh(pd[dst] + ps[src])
    e = a * d[dst] * d[src]
    return jax.ops.segment_sum(hf[src] * e[:, None], dst, num_segments=N)


def kernel(nodes, h, edge_index1, edge_index2, edge_index3, t1_w, t1_b,
           gate1_w1, gate1_b1, gate1_w2, gate1_b2, gate1_w3, gate1_b3,
           hw1_1, hw1_2, hw1_3,
           gate2_w1, gate2_b1, gate2_w2, gate2_b2, gate2_w3, gate2_b3,
           hw2_1, hw2_2, hw2_3, t2_w, t2_b, t3_w, t3_b):
    srcs = [edge_index1[0], edge_index2[0], edge_index3[0]]
    dsts = [edge_index1[1], edge_index2[1], edge_index3[1]]
    ds = [_deg_norm(dd) for dd in dsts]
    raw0 = h
    hh = _mm_lrelu(h, t1_w, t1_b)
    raw1 = hh

    g1 = [(gate1_w1, gate1_b1), (gate1_w2, gate1_b2), (gate1_w3, gate1_b3)]
    h1 = []
    for i in range(3):
        gw, gb = g1[i]
        pd = hh @ gw[:HID, 0] + gb[0]
        ps = hh @ gw[HID:, 0]
        z = _layer(hh, pd, ps, srcs[i], dsts[i], ds[i])
        h1.append(_mm_lrelu(EPS * raw1 + z, [hw1_1, hw1_2, hw1_3][i],
                            jnp.zeros((HID,), jnp.float32)))
    hh2 = jnp.concatenate(h1, axis=1)
    raw2 = hh2

    g2 = [(gate2_w1, gate2_b1), (gate2_w2, gate2_b2), (gate2_w3, gate2_b3)]
    h2 = []
    for i in range(3):
        gw, gb = g2[i]
        pd = hh2 @ gw[:3 * HID, 0] + gb[0]
        ps = hh2 @ gw[3 * HID:, 0]
        z = _layer(hh2, pd, ps, srcs[i], dsts[i], ds[i])
        h2.append(_mm_lrelu(EPS * raw2 + z, [hw2_1, hw2_2, hw2_3][i],
                            jnp.zeros((HID,), jnp.float32)))
    hh3 = jnp.concatenate(h2 + [raw0, raw1, raw2], axis=1)

    nf = hh3[nodes]
    s64 = _mm_lrelu(nf, t2_w, t2_b, block=512)
    scores = s64 @ t3_w + t3_b
    return (scores, s64)


# speedup vs baseline: 9.6829x; 9.6829x over previous
import functools

import jax
import jax.numpy as jnp
from jax import lax
from jax.experimental import pallas as pl
from jax.experimental.pallas import tpu as pltpu
from jax.experimental.pallas import tpu_sc as plsc

N = 10000
E = 320000
HID = 128
EPS = 0.3

NS = 16
NC = 2
K = 128
CH = 157
EPAD = NS * CH * K
NPAD = 10112
ROWS_PT = NPAD // NS
DUMMY = N

_MESH = plsc.VectorSubcoreMesh(core_axis_name="c", subcore_axis_name="s",
                               num_cores=NC, num_subcores=NS)


def _lrelu(x):
    return jnp.where(x >= 0, x, 0.3 * x)


def _tc_a_body(h_ref, w_ref, b_ref, g_ref, gb_ref, hh_ref, pp_ref):
    hh = _lrelu(jnp.dot(h_ref[...], w_ref[...],
                        preferred_element_type=jnp.float32) + b_ref[...])
    hh_ref[...] = hh
    pp_ref[...] = jnp.dot(hh, g_ref[...],
                          preferred_element_type=jnp.float32) + gb_ref[...]


def _tc_a(h, t1_w, t1_b, g1, gb1, block=2000):
    grid = (N // block,)
    return pl.pallas_call(
        _tc_a_body,
        grid=grid,
        in_specs=[
            pl.BlockSpec((block, HID), lambda i: (i, 0)),
            pl.BlockSpec((HID, HID), lambda i: (0, 0)),
            pl.BlockSpec((1, HID), lambda i: (0, 0)),
            pl.BlockSpec((HID, 8), lambda i: (0, 0)),
            pl.BlockSpec((1, 8), lambda i: (0, 0)),
        ],
        out_specs=[
            pl.BlockSpec((block, HID), lambda i: (i, 0)),
            pl.BlockSpec((block, 8), lambda i: (i, 0)),
        ],
        out_shape=[
            jax.ShapeDtypeStruct((N, HID), jnp.float32),
            jax.ShapeDtypeStruct((N, 8), jnp.float32),
        ],
    )(h, t1_w, t1_b.reshape(1, HID), g1, gb1.reshape(1, 8))


def _tc_d_body(deg_ref, d_ref):
    deg = deg_ref[0, :, :, 0] + deg_ref[1, :, :, 0]
    d = lax.rsqrt(deg + 1.0)
    row = lax.broadcasted_iota(jnp.int32, (3, NPAD), 1)
    d_ref[...] = jnp.where(row < N, d, 0.0)


def _tc_d(deg16):
    return pl.pallas_call(
        _tc_d_body,
        grid=(1,),
        in_specs=[pl.BlockSpec((2, 3, NPAD, 16), lambda i: (0, 0, 0, 0))],
        out_specs=pl.BlockSpec((3, NPAD), lambda i: (0, 0)),
        out_shape=jax.ShapeDtypeStruct((3, NPAD), jnp.float32),
    )(deg16)


def _tc_b_body(hh_ref, z_ref, dt_ref, hw_ref, g_ref, gb_ref,
               hh2_ref, pp_ref):
    hh = hh_ref[...]
    outs = []
    for r in range(3):
        zr = z_ref[r] * dt_ref[:, r:r + 1]
        x = EPS * hh + zr
        outs.append(_lrelu(jnp.dot(x, hw_ref[r],
                                   preferred_element_type=jnp.float32)))
    hh2 = jnp.concatenate(outs, axis=1)
    hh2_ref[...] = hh2
    pp_ref[...] = jnp.dot(hh2, g_ref[...],
                          preferred_element_type=jnp.float32) + gb_ref[...]


def _tc_b(hh, z1, dt, hw1s, g2, gb2, block=2000):
    grid = (N // block,)
    return pl.pallas_call(
        _tc_b_body,
        grid=grid,
        in_specs=[
            pl.BlockSpec((block, HID), lambda i: (i, 0)),
            pl.BlockSpec((3, block, HID), lambda i: (0, i, 0)),
            pl.BlockSpec((block, 3), lambda i: (i, 0)),
            pl.BlockSpec((3, HID, HID), lambda i: (0, 0, 0)),
            pl.BlockSpec((3 * HID, 8), lambda i: (0, 0)),
            pl.BlockSpec((1, 8), lambda i: (0, 0)),
        ],
        out_specs=[
            pl.BlockSpec((block, 3 * HID), lambda i: (i, 0)),
            pl.BlockSpec((block, 8), lambda i: (i, 0)),
        ],
        out_shape=[
            jax.ShapeDtypeStruct((N, 3 * HID), jnp.float32),
            jax.ShapeDtypeStruct((N, 8), jnp.float32),
        ],
    )(hh, z1, dt, hw1s, g2, gb2.reshape(1, 8))


def _tc_f_body(hh2_ref, z_ref, dt_ref, hw_ref, h_ref, hh_ref,
               t2w_ref, t2b_ref, y_ref):
    hh2 = hh2_ref[...]
    outs = []
    for r in range(3):
        zr = z_ref[r] * dt_ref[:, r:r + 1]
        x = EPS * hh2 + zr
        outs.append(_lrelu(jnp.dot(x, hw_ref[r],
                                   preferred_element_type=jnp.float32)))
    hh3 = jnp.concatenate(outs + [h_ref[...], hh_ref[...], hh2], axis=1)
    y_ref[...] = _lrelu(jnp.dot(hh3, t2w_ref[...],
                                preferred_element_type=jnp.float32)
                        + t2b_ref[...])


def _tc_f(hh2, z2, dt, hw2s, h, hh, t2_w, t2_b, block=1000):
    grid = (N // block,)
    return pl.pallas_call(
        _tc_f_body,
        grid=grid,
        in_specs=[
            pl.BlockSpec((block, 3 * HID), lambda i: (i, 0)),
            pl.BlockSpec((3, block, 3 * HID), lambda i: (0, i, 0)),
            pl.BlockSpec((block, 3), lambda i: (i, 0)),
            pl.BlockSpec((3, 3 * HID, HID), lambda i: (0, 0, 0)),
            pl.BlockSpec((block, HID), lambda i: (i, 0)),
            pl.BlockSpec((block, HID), lambda i: (i, 0)),
            pl.BlockSpec((7 * HID + HID, 64), lambda i: (0, 0)),
            pl.BlockSpec((1, 64), lambda i: (0, 0)),
        ],
        out_specs=pl.BlockSpec((block, 64), lambda i: (i, 0)),
        out_shape=jax.ShapeDtypeStruct((N, 64), jnp.float32),
    )(hh2, z2, dt, hw2s, h, hh, t2_w, t2_b.reshape(1, 64))


def _tc_e_body(s_ref, w_ref, b_ref, o_ref):
    o_ref[...] = jnp.dot(s_ref[...], w_ref[...],
                         preferred_element_type=jnp.float32) + b_ref[...]


def _tc_e(s64, t3_w, t3_b):
    return pl.pallas_call(
        _tc_e_body,
        grid=(1,),
        in_specs=[
            pl.BlockSpec((4096, 64), lambda i: (0, 0)),
            pl.BlockSpec((64, 2), lambda i: (0, 0)),
            pl.BlockSpec((1, 2), lambda i: (0, 0)),
        ],
        out_specs=pl.BlockSpec((4096, 2), lambda i: (0, 0)),
        out_shape=jax.ShapeDtypeStruct((4096, 2), jnp.float32),
    )(s64, t3_w, t3_b.reshape(1, 2))


def _make_deg():
    @functools.partial(
        pl.kernel,
        out_type=jax.ShapeDtypeStruct((NC * 3 * NPAD, 16), jnp.float32),
        mesh=_MESH,
        compiler_params=pltpu.CompilerParams(needs_layout_passes=False,
                                             use_tc_tiling_on_sc=False),
        scratch_types=[
            pltpu.VMEM((K,), jnp.int32),
            pltpu.VMEM((K, 16), jnp.float32),
            pltpu.VMEM_SHARED((NPAD, 16), jnp.float32),
        ],
    )
    def deg_kernel(dst_hbm, ones_hbm, zeros_hbm, out_hbm,
                   dst_chunk, ones_v, acc):
        c = lax.axis_index("c")
        s = lax.axis_index("s")
        pltpu.sync_copy(ones_hbm, ones_v)
        ch_lo = c * 79
        ch_n = 79 - c

        def rel(r, _):
            pltpu.sync_copy(zeros_hbm.at[pl.ds(s * ROWS_PT, ROWS_PT)],
                            acc.at[pl.ds(s * ROWS_PT, ROWS_PT)])
            plsc.subcore_barrier()
            ebase = (r * NS + s) * (CH * K)

            def chunk(i, _):
                pltpu.sync_copy(
                    dst_hbm.at[pl.ds(ebase + (ch_lo + i) * K, K)],
                    dst_chunk)
                pltpu.sync_copy(ones_v, acc.at[dst_chunk], add=True)
                return 0

            lax.fori_loop(0, ch_n, chunk, 0)
            plsc.subcore_barrier()
            base = (c * 3 + r) * NPAD + s * ROWS_PT
            pltpu.sync_copy(acc.at[pl.ds(s * ROWS_PT, ROWS_PT)],
                            out_hbm.at[pl.ds(base, ROWS_PT)])
            plsc.subcore_barrier()
            return 0

        lax.fori_loop(0, 3, rel, 0)

    return deg_kernel


def _make_agg(W, FP):
    NSLOT = NC * FP

    @functools.partial(
        pl.kernel,
        out_type=jax.ShapeDtypeStruct((3 * NSLOT * NPAD, W), jnp.float32),
        mesh=_MESH,
        compiler_params=pltpu.CompilerParams(needs_layout_passes=False,
                                             use_tc_tiling_on_sc=False),
        scratch_types=[
            pltpu.VMEM((K,), jnp.int32),
            pltpu.VMEM((K,), jnp.int32),
            pltpu.VMEM((K,), jnp.int32),
            pltpu.VMEM((K,), jnp.float32),
            pltpu.VMEM((NPAD,), jnp.float32),
            pltpu.VMEM((NPAD,), jnp.float32),
            pltpu.VMEM((NPAD,), jnp.float32),
            pltpu.VMEM((K, W), jnp.float32),
            pltpu.VMEM_SHARED((NPAD, W), jnp.float32),
            pltpu.SemaphoreType.DMA,
        ],
    )
    def agg_kernel(src_hbm, dst_hbm, pd_hbm, ps_hbm, d_hbm, table_hbm,
                   zeros_hbm, out_hbm,
                   src_chunk, dst_chunk, gsrc, f_buf,
                   pd_t, ps_t, d_t, rows, acc, gsem):
        c = lax.axis_index("c")
        s = lax.axis_index("s")

        def rp_body(rp, _):
            r = rp // FP
            p = rp - r * FP
            slot = c * FP + p
            pltpu.sync_copy(pd_hbm.at[r], pd_t)
            pltpu.sync_copy(ps_hbm.at[r], ps_t)
            pltpu.sync_copy(d_hbm.at[r], d_t)
            pltpu.sync_copy(zeros_hbm.at[pl.ds(s * ROWS_PT, ROWS_PT)],
                            acc.at[pl.ds(s * ROWS_PT, ROWS_PT)])
            plsc.subcore_barrier()
            ebase = (r * NS + s) * (CH * K)

            def chunk(ch, _):
                pltpu.sync_copy(src_hbm.at[pl.ds(ebase + ch * K, K)],
                                src_chunk)
                pltpu.sync_copy(dst_hbm.at[pl.ds(ebase + ch * K, K)],
                                dst_chunk)
                off = slot * NPAD
                for j in range(K // 16):
                    sl = pl.ds(j * 16, 16)
                    gsrc[sl] = src_chunk[sl] + off
                cp = pltpu.async_copy(table_hbm.at[gsrc], rows, gsem)
                for j in range(K // 16):
                    sl = pl.ds(j * 16, 16)
                    sv = src_chunk[sl]
                    dv = dst_chunk[sl]
                    pd = plsc.load_gather(pd_t, [dv])
                    ps = plsc.load_gather(ps_t, [sv])
                    dd = plsc.load_gather(d_t, [sv])
                    x = pd + ps
                    t = jnp.exp(-2.0 * jnp.abs(x))
                    a = (1.0 - t) / (1.0 + t)
                    f_buf[sl] = jnp.where(x < 0.0, -a, a) * dd
                cp.wait()
                for j in range(K // 16):
                    fv = f_buf[pl.ds(j * 16, 16)]
                    for k16 in range(16):
                        k = j * 16 + k16
                        fk = fv[k16]
                        for q in range(W // 16):
                            sl = pl.ds(q * 16, 16)
                            rows[k, sl] = rows[k, sl] * fk
                pltpu.sync_copy(rows, acc.at[dst_chunk], add=True)
                return 0

            lax.fori_loop(0, CH, chunk, 0)
            plsc.subcore_barrier()
            base = (r * NSLOT + slot) * NPAD + s * ROWS_PT
            pltpu.sync_copy(acc.at[pl.ds(s * ROWS_PT, ROWS_PT)],
                            out_hbm.at[pl.ds(base, ROWS_PT)])
            plsc.subcore_barrier()
            return 0

        lax.fori_loop(0, 3 * FP, rp_body, 0)

    return agg_kernel


def _make_gather():
    @functools.partial(
        pl.kernel,
        out_type=jax.ShapeDtypeStruct((4096, 64), jnp.float32),
        mesh=_MESH,
        compiler_params=pltpu.CompilerParams(needs_layout_passes=False,
                                             use_tc_tiling_on_sc=False),
        scratch_types=[
            pltpu.VMEM((K,), jnp.int32),
            pltpu.VMEM((K, 64), jnp.float32),
            pltpu.SemaphoreType.DMA,
        ],
    )
    def gather_kernel(y_hbm, nodes_hbm, out_hbm, idx, rows, sem):
        c = lax.axis_index("c")
        s = lax.axis_index("s")
        w = s * NC + c
        base = w * K
        pltpu.sync_copy(nodes_hbm.at[pl.ds(base, K)], idx)
        pltpu.async_copy(y_hbm.at[idx], rows, sem).wait()
        pltpu.sync_copy(rows, out_hbm.at[pl.ds(base, K)])

    return gather_kernel


_DEG = _make_deg()
_AGG1 = _make_agg(64, 1)
_AGG2 = _make_agg(96, 2)
_GATHER = _make_gather()


def _pad_edges(ei):
    pad = jnp.full((EPAD - E,), DUMMY, jnp.int32)
    src = jnp.concatenate([ei[0], pad])
    dst = jnp.concatenate([ei[1], pad])
    return src, dst


def _pad_rows(x):
    return jnp.pad(x, ((0, NPAD - N), (0, 0)))


def _gate_mats(ws, bs, dim):
    cols = []
    for w, _ in zip(ws, bs):
        cols.append(w[:dim, 0])
        cols.append(w[dim:, 0])
    cols.append(jnp.zeros((dim,), jnp.float32))
    cols.append(jnp.zeros((dim,), jnp.float32))
    g = jnp.stack(cols, axis=1)
    gb = jnp.stack([bs[0][0], jnp.float32(0), bs[1][0], jnp.float32(0),
                    bs[2][0], jnp.float32(0), jnp.float32(0),
                    jnp.float32(0)])
    return g, gb


def _split_pdps(pp):
    pd = jnp.pad(jnp.stack([pp[:, 0], pp[:, 2], pp[:, 4]]),
                 ((0, 0), (0, NPAD - N)))
    ps = jnp.pad(jnp.stack([pp[:, 1], pp[:, 3], pp[:, 5]]),
                 ((0, 0), (0, NPAD - N)))
    return pd, ps


def kernel(nodes, h, edge_index1, edge_index2, edge_index3, t1_w, t1_b,
           gate1_w1, gate1_b1, gate1_w2, gate1_b2, gate1_w3, gate1_b3,
           hw1_1, hw1_2, hw1_3,
           gate2_w1, gate2_b1, gate2_w2, gate2_b2, gate2_w3, gate2_b3,
           hw2_1, hw2_2, hw2_3, t2_w, t2_b, t3_w, t3_b):
    e1s, e1d = _pad_edges(edge_index1)
    e2s, e2d = _pad_edges(edge_index2)
    e3s, e3d = _pad_edges(edge_index3)
    src3 = jnp.concatenate([e1s, e2s, e3s])
    dst3 = jnp.concatenate([e1d, e2d, e3d])

    ones16 = jnp.ones((K, 16), jnp.float32)
    z16 = jnp.zeros((NPAD, 16), jnp.float32)
    z64 = jnp.zeros((NPAD, 64), jnp.float32)
    z96 = jnp.zeros((NPAD, 96), jnp.float32)

    deg16 = _DEG(dst3, ones16, z16).reshape(NC, 3, NPAD, 16)
    d3 = _tc_d(deg16)
    dt = d3[:, :N].T

    g1, gb1 = _gate_mats([gate1_w1, gate1_w2, gate1_w3],
                         [gate1_b1, gate1_b2, gate1_b3], HID)
    hh, pp1 = _tc_a(h, t1_w, t1_b, g1, gb1)
    pd1, ps1 = _split_pdps(pp1)

    hhp = _pad_rows(hh)
    table1 = jnp.concatenate([hhp[:, :64], hhp[:, 64:]], axis=0)
    zr1 = _AGG1(src3, dst3, pd1, ps1, d3, table1, z64)
    z1 = (zr1.reshape(3, NC, NPAD, 64)[:, :, :N, :]
          .transpose(0, 2, 1, 3).reshape(3, N, HID))

    g2, gb2 = _gate_mats([gate2_w1, gate2_w2, gate2_w3],
                         [gate2_b1, gate2_b2, gate2_b3], 3 * HID)
    hw1s = jnp.stack([hw1_1, hw1_2, hw1_3])
    hh2, pp2 = _tc_b(hh, z1, dt, hw1s, g2, gb2)
    pd2, ps2 = _split_pdps(pp2)

    hh2p = _pad_rows(hh2)
    table2 = jnp.concatenate([hh2p[:, 0:96], hh2p[:, 96:192],
                              hh2p[:, 192:288], hh2p[:, 288:384]], axis=0)
    zr2 = _AGG2(src3, dst3, pd2, ps2, d3, table2, z96)
    z2 = (zr2.reshape(3, 4, NPAD, 96)[:, :, :N, :]
          .transpose(0, 2, 1, 3).reshape(3, N, 3 * HID))

    hw2s = jnp.stack([hw2_1, hw2_2, hw2_3])
    y = _tc_f(hh2, z2, dt, hw2s, h, hh, t2_w, t2_b)

    s64 = _GATHER(y, nodes)
    scores = _tc_e(s64, t3_w, t3_b)
    return (scores, s64)


# speedup vs baseline: 9.7116x; 1.0030x over previous
import functools

import jax
import jax.numpy as jnp
from jax import lax
from jax.experimental import pallas as pl
from jax.experimental.pallas import tpu as pltpu
from jax.experimental.pallas import tpu_sc as plsc

N = 10000
E = 320000
HID = 128
EPS = 0.3

NS = 16
NC = 2
K = 128
CHT = 160
EPAD = NS * CHT * K
NPAD = 10112
ROWS_PT = NPAD // NS
DUMMY = N
NW = NC * NS
W = 64

_MESH = plsc.VectorSubcoreMesh(core_axis_name="c", subcore_axis_name="s",
                               num_cores=NC, num_subcores=NS)
_SC_PARAMS = pltpu.CompilerParams(needs_layout_passes=False,
                                  use_tc_tiling_on_sc=False)


def _lrelu(x):
    return jnp.where(x >= 0, x, 0.3 * x)


def _tc_a_body(h_ref, w_ref, b_ref, g_ref, gb_ref, hh_ref, pp_ref):
    hh = _lrelu(jnp.dot(h_ref[...], w_ref[...],
                        preferred_element_type=jnp.float32) + b_ref[...])
    hh_ref[...] = hh
    pp_ref[...] = jnp.dot(hh, g_ref[...],
                          preferred_element_type=jnp.float32) + gb_ref[...]


def _tc_a(h, t1_w, t1_b, g1, gb1, block=2000):
    grid = (N // block,)
    return pl.pallas_call(
        _tc_a_body,
        grid=grid,
        in_specs=[
            pl.BlockSpec((block, HID), lambda i: (i, 0)),
            pl.BlockSpec((HID, HID), lambda i: (0, 0)),
            pl.BlockSpec((1, HID), lambda i: (0, 0)),
            pl.BlockSpec((HID, 8), lambda i: (0, 0)),
            pl.BlockSpec((1, 8), lambda i: (0, 0)),
        ],
        out_specs=[
            pl.BlockSpec((block, HID), lambda i: (i, 0)),
            pl.BlockSpec((block, 8), lambda i: (i, 0)),
        ],
        out_shape=[
            jax.ShapeDtypeStruct((N, HID), jnp.float32),
            jax.ShapeDtypeStruct((N, 8), jnp.float32),
        ],
    )(h, t1_w, t1_b.reshape(1, HID), g1, gb1.reshape(1, 8))


def _tc_d_body(deg_ref, d_ref):
    deg = jnp.sum(deg_ref[...], axis=1)
    d = lax.rsqrt(deg + 1.0)
    row = lax.broadcasted_iota(jnp.int32, (3, NPAD), 1)
    d_ref[...] = jnp.where(row < N, d, 0.0)


def _tc_d(degp):
    return pl.pallas_call(
        _tc_d_body,
        grid=(1,),
        in_specs=[pl.BlockSpec((3, NW, NPAD), lambda i: (0, 0, 0))],
        out_specs=pl.BlockSpec((3, NPAD), lambda i: (0, 0)),
        out_shape=jax.ShapeDtypeStruct((3, NPAD), jnp.float32),
    )(degp)


def _tc_b_body(hh_ref, z_ref, dt_ref, hw_ref, g_ref, gb_ref,
               hh2_ref, pp_ref):
    hh = hh_ref[...]
    outs = []
    for r in range(3):
        zr = z_ref[r] * dt_ref[:, r:r + 1]
        x = EPS * hh + zr
        outs.append(_lrelu(jnp.dot(x, hw_ref[r],
                                   preferred_element_type=jnp.float32)))
    hh2 = jnp.concatenate(outs, axis=1)
    hh2_ref[...] = hh2
    pp_ref[...] = jnp.dot(hh2, g_ref[...],
                          preferred_element_type=jnp.float32) + gb_ref[...]


def _tc_b(hh, z1, dt, hw1s, g2, gb2, block=2000):
    grid = (N // block,)
    return pl.pallas_call(
        _tc_b_body,
        grid=grid,
        in_specs=[
            pl.BlockSpec((block, HID), lambda i: (i, 0)),
            pl.BlockSpec((3, block, HID), lambda i: (0, i, 0)),
            pl.BlockSpec((block, 3), lambda i: (i, 0)),
            pl.BlockSpec((3, HID, HID), lambda i: (0, 0, 0)),
            pl.BlockSpec((3 * HID, 8), lambda i: (0, 0)),
            pl.BlockSpec((1, 8), lambda i: (0, 0)),
        ],
        out_specs=[
            pl.BlockSpec((block, 3 * HID), lambda i: (i, 0)),
            pl.BlockSpec((block, 8), lambda i: (i, 0)),
        ],
        out_shape=[
            jax.ShapeDtypeStruct((N, 3 * HID), jnp.float32),
            jax.ShapeDtypeStruct((N, 8), jnp.float32),
        ],
    )(hh, z1, dt, hw1s, g2, gb2.reshape(1, 8))


def _tc_f_body(hh2_ref, z_ref, dt_ref, hw_ref, h_ref, hh_ref,
               t2w_ref, t2b_ref, y_ref):
    hh2 = hh2_ref[...]
    outs = []
    for r in range(3):
        x = EPS * hh2 + z_ref[r] * dt_ref[:, r:r + 1]
        outs.append(_lrelu(jnp.dot(x, hw_ref[r],
                                   preferred_element_type=jnp.float32)))
    hh3 = jnp.concatenate(outs + [h_ref[...], hh_ref[...], hh2], axis=1)
    y_ref[...] = _lrelu(jnp.dot(hh3, t2w_ref[...],
                                preferred_element_type=jnp.float32)
                        + t2b_ref[...])


def _tc_f(hh2, z2, dt, hw2s, h, hh, t2_w, t2_b, block=1000):
    grid = (N // block,)
    return pl.pallas_call(
        _tc_f_body,
        grid=grid,
        in_specs=[
            pl.BlockSpec((block, 3 * HID), lambda i: (i, 0)),
            pl.BlockSpec((3, block, 3 * HID), lambda i: (0, i, 0)),
            pl.BlockSpec((block, 3), lambda i: (i, 0)),
            pl.BlockSpec((3, 3 * HID, HID), lambda i: (0, 0, 0)),
            pl.BlockSpec((block, HID), lambda i: (i, 0)),
            pl.BlockSpec((block, HID), lambda i: (i, 0)),
            pl.BlockSpec((7 * HID + HID, 64), lambda i: (0, 0)),
            pl.BlockSpec((1, 64), lambda i: (0, 0)),
        ],
        out_specs=pl.BlockSpec((block, 64), lambda i: (i, 0)),
        out_shape=jax.ShapeDtypeStruct((N, 64), jnp.float32),
    )(hh2, z2, dt, hw2s, h, hh, t2_w, t2_b.reshape(1, 64))


def _tc_e_body(s_ref, w_ref, b_ref, o_ref):
    o_ref[...] = jnp.dot(s_ref[...], w_ref[...],
                         preferred_element_type=jnp.float32) + b_ref[...]


def _tc_e(s64, t3_w, t3_b):
    return pl.pallas_call(
        _tc_e_body,
        grid=(1,),
        in_specs=[
            pl.BlockSpec((4096, 64), lambda i: (0, 0)),
            pl.BlockSpec((64, 2), lambda i: (0, 0)),
            pl.BlockSpec((1, 2), lambda i: (0, 0)),
        ],
        out_specs=pl.BlockSpec((4096, 2), lambda i: (0, 0)),
        out_shape=jax.ShapeDtypeStruct((4096, 2), jnp.float32),
    )(s64, t3_w, t3_b.reshape(1, 2))


def _make_deg():
    CHC = CHT // 2

    @functools.partial(
        pl.kernel,
        out_type=jax.ShapeDtypeStruct((3 * NW * NPAD,), jnp.float32),
        mesh=_MESH,
        compiler_params=_SC_PARAMS,
        scratch_types=[
            pltpu.VMEM((2, K), jnp.int32),
            pltpu.VMEM((NPAD,), jnp.float32),
        ],
    )
    def deg_kernel(sd_hbm, out_hbm, sd1, deg_t):
        c = lax.axis_index("c")
        s = lax.axis_index("s")
        ones = jnp.full((16,), 1.0, jnp.float32)
        zeros = jnp.zeros((16,), jnp.float32)

        def rel(r, _):
            def zrow(i, _):
                deg_t[pl.ds(i * 16, 16)] = zeros
                return 0

            lax.fori_loop(0, NPAD // 16, zrow, 0)
            cbase = (r * NS + s) * CHT + c * CHC

            def chunk(i, _):
                pltpu.sync_copy(sd_hbm.at[pl.ds(2 * (cbase + i), 2)], sd1)
                for jj in range(K // 16):
                    dv = sd1[1, pl.ds(jj * 16, 16)]
                    plsc.addupdate_scatter(deg_t, [dv], ones)
                return 0

            lax.fori_loop(0, CHC, chunk, 0)
            base = ((r * NC + c) * NS + s) * NPAD
            pltpu.sync_copy(deg_t, out_hbm.at[pl.ds(base, NPAD)])
            return 0

        lax.fori_loop(0, 3, rel, 0)

    return deg_kernel


def _make_agg(FP, SUP):
    NSLOT = NC * FP
    G = CHT // SUP
    assert G % 2 == 0

    @functools.partial(
        pl.kernel,
        out_type=jax.ShapeDtypeStruct((3 * NSLOT * NPAD, W), jnp.float32),
        mesh=_MESH,
        compiler_params=_SC_PARAMS,
        scratch_types=[
            pltpu.VMEM((2 * SUP, K), jnp.int32),
            pltpu.VMEM((2 * SUP, K), jnp.int32),
            pltpu.VMEM((SUP, K), jnp.int32),
            pltpu.VMEM((SUP, K), jnp.int32),
            pltpu.VMEM((SUP * K, W), jnp.float32),
            pltpu.VMEM((SUP * K, W), jnp.float32),
            pltpu.VMEM((NPAD,), jnp.float32),
            pltpu.VMEM((NPAD,), jnp.float32),
            pltpu.VMEM((NPAD,), jnp.float32),
            pltpu.VMEM_SHARED((NPAD, W), jnp.float32),
            pltpu.SemaphoreType.DMA,
            pltpu.SemaphoreType.DMA,
            pltpu.SemaphoreType.DMA,
            pltpu.SemaphoreType.DMA,
        ],
    )
    def agg_kernel(sd_hbm, pd_hbm, ps_hbm, d_hbm, table_hbm,
                   zeros_hbm, out_hbm,
                   sd2a, sd2b, gsrca, gsrcb, rowsa, rowsb,
                   pd_t, ps_t, d_t, acc, gsem0, gsem1, osem0, osem1):
        c = lax.axis_index("c")
        s = lax.axis_index("s")
        sd2 = (sd2a, sd2b)
        gsrc = (gsrca, gsrcb)
        rows = (rowsa, rowsb)
        gsem = (gsem0, gsem1)
        osem = (osem0, osem1)

        def drain(b, sem):
            pltpu.make_async_copy(zeros_hbm.at[pl.ds(0, K)],
                                  rows[b].at[pl.ds(0, K)], sem).wait()

        def front(b, g, cbase, toff, do_drain):
            if do_drain:
                for _ in range(SUP):
                    drain(b, osem[b])
            pltpu.sync_copy(sd_hbm.at[pl.ds(2 * (cbase + g * SUP), 2 * SUP)],
                            sd2[b])
            for j in range(SUP):
                for jj in range(K // 16):
                    sl = pl.ds(jj * 16, 16)
                    gsrc[b][j, sl] = sd2[b][2 * j, sl] + toff
            for j in range(SUP):
                pltpu.async_copy(table_hbm.at[gsrc[b].at[j]],
                                 rows[b].at[pl.ds(j * K, K)], gsem[b])

        def back(b):
            for _ in range(SUP):
                drain(b, gsem[b])

            def chunk(j, _):
                for jj in range(K // 16):
                    sl = pl.ds(jj * 16, 16)
                    sv = sd2[b][2 * j, sl]
                    dv = sd2[b][2 * j + 1, sl]
                    pd = plsc.load_gather(pd_t, [dv])
                    ps = plsc.load_gather(ps_t, [sv])
                    dd = plsc.load_gather(d_t, [sv])
                    x = pd + ps
                    t = jnp.exp(-2.0 * jnp.abs(x))
                    a = (1.0 - t) / (1.0 + t)
                    f = jnp.where(x < 0.0, -a, a) * dd
                    for k16 in range(16):
                        fk = f[k16]
                        row = rows[b].at[j * K + jj * 16 + k16]
                        for q in range(W // 16):
                            sl2 = pl.ds(q * 16, 16)
                            row[sl2] = row[sl2] * fk
                pltpu.async_copy(rows[b].at[pl.ds(j * K, K)],
                                 acc.at[sd2[b].at[2 * j + 1]], osem[b],
                                 add=True)
                return 0

            lax.fori_loop(0, SUP, chunk, 0)

        def rp_body(rp, _):
            r = rp // FP
            p = rp - r * FP
            slot = c * FP + p
            toff = slot * NPAD
            pltpu.sync_copy(pd_hbm.at[pl.ds(r * NPAD, NPAD)], pd_t)
            pltpu.sync_copy(ps_hbm.at[pl.ds(r * NPAD, NPAD)], ps_t)
            pltpu.sync_copy(d_hbm.at[pl.ds(r * NPAD, NPAD)], d_t)
            pltpu.sync_copy(zeros_hbm.at[pl.ds(s * ROWS_PT, ROWS_PT)],
                            acc.at[pl.ds(s * ROWS_PT, ROWS_PT)])
            plsc.subcore_barrier()
            cbase = (r * NS + s) * CHT
            front(0, 0, cbase, toff, False)
            front(1, 1, cbase, toff, False)

            def pair(i, _):
                back(0)
                back(1)
                g2 = 2 * i + 2

                @pl.when(g2 < G)
                def _():
                    front(0, g2, cbase, toff, True)
                    front(1, g2 + 1, cbase, toff, True)

                return 0

            lax.fori_loop(0, G // 2, pair, 0)
            for b in range(2):
                for _ in range(SUP):
                    drain(b, osem[b])
            plsc.subcore_barrier()
            base = (r * NSLOT + slot) * NPAD + s * ROWS_PT
            pltpu.sync_copy(acc.at[pl.ds(s * ROWS_PT, ROWS_PT)],
                            out_hbm.at[pl.ds(base, ROWS_PT)])
            plsc.subcore_barrier()
            return 0

        lax.fori_loop(0, 3 * FP, rp_body, 0)

    return agg_kernel


def _make_gather():
    @functools.partial(
        pl.kernel,
        out_type=jax.ShapeDtypeStruct((4096, 64), jnp.float32),
        mesh=_MESH,
        compiler_params=_SC_PARAMS,
        scratch_types=[
            pltpu.VMEM((K,), jnp.int32),
            pltpu.VMEM((K, 64), jnp.float32),
            pltpu.SemaphoreType.DMA,
        ],
    )
    def gather_kernel(y_hbm, nodes_hbm, out_hbm, idx, rows, sem):
        c = lax.axis_index("c")
        s = lax.axis_index("s")
        w = s * NC + c
        base = w * K
        pltpu.sync_copy(nodes_hbm.at[pl.ds(base, K)], idx)
        pltpu.async_copy(y_hbm.at[idx], rows, sem).wait()
        pltpu.sync_copy(rows, out_hbm.at[pl.ds(base, K)])

    return gather_kernel


_DEG = _make_deg()
_AGG1 = _make_agg(1, 2)
_AGG2 = _make_agg(3, 2)
_GATHER = _make_gather()


def _pad_edges(ei):
    pad = jnp.full((EPAD - E,), DUMMY, jnp.int32)
    src = jnp.concatenate([ei[0], pad]).reshape(-1, K)
    dst = jnp.concatenate([ei[1], pad]).reshape(-1, K)
    return jnp.stack([src, dst], axis=1).reshape(-1, K)


def _pad_rows(x):
    return jnp.pad(x, ((0, NPAD - N), (0, 0)))


def _gate_mats(ws, bs, dim):
    cols = []
    for w_, _ in zip(ws, bs):
        cols.append(w_[:dim, 0])
        cols.append(w_[dim:, 0])
    cols.append(jnp.zeros((dim,), jnp.float32))
    cols.append(jnp.zeros((dim,), jnp.float32))
    g = jnp.stack(cols, axis=1)
    gb = jnp.stack([bs[0][0], jnp.float32(0), bs[1][0], jnp.float32(0),
                    bs[2][0], jnp.float32(0), jnp.float32(0),
                    jnp.float32(0)])
    return g, gb


def _split_pdps(pp):
    pd = jnp.pad(jnp.stack([pp[:, 0], pp[:, 2], pp[:, 4]]),
                 ((0, 0), (0, NPAD - N))).reshape(-1)
    ps = jnp.pad(jnp.stack([pp[:, 1], pp[:, 3], pp[:, 5]]),
                 ((0, 0), (0, NPAD - N))).reshape(-1)
    return pd, ps


def _slices64(x, nslices):
    return jnp.concatenate([x[:, W * q:W * (q + 1)] for q in range(nslices)],
                           axis=0)


def kernel(nodes, h, edge_index1, edge_index2, edge_index3, t1_w, t1_b,
           gate1_w1, gate1_b1, gate1_w2, gate1_b2, gate1_w3, gate1_b3,
           hw1_1, hw1_2, hw1_3,
           gate2_w1, gate2_b1, gate2_w2, gate2_b2, gate2_w3, gate2_b3,
           hw2_1, hw2_2, hw2_3, t2_w, t2_b, t3_w, t3_b):
    sd3 = jnp.concatenate([_pad_edges(edge_index1),
                           _pad_edges(edge_index2),
                           _pad_edges(edge_index3)])

    z64 = jnp.zeros((NPAD, W), jnp.float32)

    degp = _DEG(sd3).reshape(3, NW, NPAD)
    d3 = _tc_d(degp)
    dt = d3[:, :N].T
    d3f = d3.reshape(-1)

    g1, gb1 = _gate_mats([gate1_w1, gate1_w2, gate1_w3],
                         [gate1_b1, gate1_b2, gate1_b3], HID)
    hh, pp1 = _tc_a(h, t1_w, t1_b, g1, gb1)
    pd1, ps1 = _split_pdps(pp1)

    table1 = _slices64(_pad_rows(hh), 2)
    zr1 = _AGG1(sd3, pd1, ps1, d3f, table1, z64)
    z1 = (zr1.reshape(3, 2, NPAD, W).transpose(0, 2, 1, 3)
          .reshape(3, NPAD, HID)[:, :N, :])

    g2, gb2 = _gate_mats([gate2_w1, gate2_w2, gate2_w3],
                         [gate2_b1, gate2_b2, gate2_b3], 3 * HID)
    hw1s = jnp.stack([hw1_1, hw1_2, hw1_3])
    hh2, pp2 = _tc_b(hh, z1, dt, hw1s, g2, gb2)
    pd2, ps2 = _split_pdps(pp2)

    table2 = _slices64(_pad_rows(hh2), 6)
    zr2 = _AGG2(sd3, pd2, ps2, d3f, table2, z64)
    z2 = (zr2.reshape(3, 6, NPAD, W).transpose(0, 2, 1, 3)
          .reshape(3, NPAD, 3 * HID)[:, :N, :])

    hw2s = jnp.stack([hw2_1, hw2_2, hw2_3])
    y = _tc_f(hh2, z2, dt, hw2s, h, hh, t2_w, t2_b)

    s64 = _GATHER(y, nodes)
    scores = _tc_e(s64, t3_w, t3_b)
    return (scores, s64)


# speedup vs baseline: 9.7154x; 1.0004x over previous
import functools

import jax
import jax.numpy as jnp
from jax import lax
from jax.experimental import pallas as pl
from jax.experimental.pallas import tpu as pltpu
from jax.experimental.pallas import tpu_sc as plsc

N = 10000
E = 320000
HID = 128
EPS = 0.3

NS = 16
NC = 2
K = 128
CHT = 160
EPAD = NS * CHT * K
NPAD = 10112
ROWS_PT = NPAD // NS
DUMMY = N
NW = NC * NS
W = 64

_MESH = plsc.VectorSubcoreMesh(core_axis_name="c", subcore_axis_name="s",
                               num_cores=NC, num_subcores=NS)
_SC_PARAMS = pltpu.CompilerParams(needs_layout_passes=False,
                                  use_tc_tiling_on_sc=False)


def _lrelu(x):
    return jnp.where(x >= 0, x, 0.3 * x)


def _tc_a_body(h_ref, w_ref, b_ref, g_ref, gb_ref, hh_ref, pp_ref):
    hh = _lrelu(jnp.dot(h_ref[...], w_ref[...],
                        preferred_element_type=jnp.float32) + b_ref[...])
    hh_ref[...] = hh
    pp_ref[...] = jnp.dot(hh, g_ref[...],
                          preferred_element_type=jnp.float32) + gb_ref[...]


def _tc_a(h, t1_w, t1_b, g1, gb1, block=2000):
    grid = (N // block,)
    return pl.pallas_call(
        _tc_a_body,
        grid=grid,
        in_specs=[
            pl.BlockSpec((block, HID), lambda i: (i, 0)),
            pl.BlockSpec((HID, HID), lambda i: (0, 0)),
            pl.BlockSpec((1, HID), lambda i: (0, 0)),
            pl.BlockSpec((HID, 8), lambda i: (0, 0)),
            pl.BlockSpec((1, 8), lambda i: (0, 0)),
        ],
        out_specs=[
            pl.BlockSpec((block, HID), lambda i: (i, 0)),
            pl.BlockSpec((block, 8), lambda i: (i, 0)),
        ],
        out_shape=[
            jax.ShapeDtypeStruct((N, HID), jnp.float32),
            jax.ShapeDtypeStruct((N, 8), jnp.float32),
        ],
    )(h, t1_w, t1_b.reshape(1, HID), g1, gb1.reshape(1, 8))


def _tc_d_body(deg_ref, d_ref):
    deg = jnp.sum(deg_ref[...], axis=1)
    d = lax.rsqrt(deg + 1.0)
    row = lax.broadcasted_iota(jnp.int32, (3, NPAD), 1)
    d_ref[...] = jnp.where(row < N, d, 0.0)


def _tc_d(degp):
    return pl.pallas_call(
        _tc_d_body,
        grid=(1,),
        in_specs=[pl.BlockSpec((3, NW, NPAD), lambda i: (0, 0, 0))],
        out_specs=pl.BlockSpec((3, NPAD), lambda i: (0, 0)),
        out_shape=jax.ShapeDtypeStruct((3, NPAD), jnp.float32),
    )(degp)


def _tc_b_body(hh_ref, z_ref, dt_ref, hw_ref, g_ref, gb_ref,
               hh2_ref, pp_ref):
    hh = hh_ref[...]
    outs = []
    for r in range(3):
        zr = z_ref[r] * dt_ref[:, r:r + 1]
        x = EPS * hh + zr
        outs.append(_lrelu(jnp.dot(x, hw_ref[r],
                                   preferred_element_type=jnp.float32)))
    hh2 = jnp.concatenate(outs, axis=1)
    hh2_ref[...] = hh2
    pp_ref[...] = jnp.dot(hh2, g_ref[...],
                          preferred_element_type=jnp.float32) + gb_ref[...]


def _tc_b(hh, z1, dt, hw1s, g2, gb2, block=2000):
    grid = (N // block,)
    return pl.pallas_call(
        _tc_b_body,
        grid=grid,
        in_specs=[
            pl.BlockSpec((block, HID), lambda i: (i, 0)),
            pl.BlockSpec((3, block, HID), lambda i: (0, i, 0)),
            pl.BlockSpec((block, 3), lambda i: (i, 0)),
            pl.BlockSpec((3, HID, HID), lambda i: (0, 0, 0)),
            pl.BlockSpec((3 * HID, 8), lambda i: (0, 0)),
            pl.BlockSpec((1, 8), lambda i: (0, 0)),
        ],
        out_specs=[
            pl.BlockSpec((block, 3 * HID), lambda i: (i, 0)),
            pl.BlockSpec((block, 8), lambda i: (i, 0)),
        ],
        out_shape=[
            jax.ShapeDtypeStruct((N, 3 * HID), jnp.float32),
            jax.ShapeDtypeStruct((N, 8), jnp.float32),
        ],
    )(hh, z1, dt, hw1s, g2, gb2.reshape(1, 8))


def _tc_f_body(hh2_ref, z_ref, dt_ref, hw_ref, h_ref, hh_ref,
               t2w_ref, t2b_ref, y_ref):
    hh2 = hh2_ref[...]
    outs = []
    for r in range(3):
        x = EPS * hh2 + z_ref[r] * dt_ref[:, r:r + 1]
        outs.append(_lrelu(jnp.dot(x, hw_ref[r],
                                   preferred_element_type=jnp.float32)))
    hh3 = jnp.concatenate(outs + [h_ref[...], hh_ref[...], hh2], axis=1)
    y_ref[...] = _lrelu(jnp.dot(hh3, t2w_ref[...],
                                preferred_element_type=jnp.float32)
                        + t2b_ref[...])


def _tc_f(hh2, z2, dt, hw2s, h, hh, t2_w, t2_b, block=1000):
    grid = (N // block,)
    return pl.pallas_call(
        _tc_f_body,
        grid=grid,
        in_specs=[
            pl.BlockSpec((block, 3 * HID), lambda i: (i, 0)),
            pl.BlockSpec((3, block, 3 * HID), lambda i: (0, i, 0)),
            pl.BlockSpec((block, 3), lambda i: (i, 0)),
            pl.BlockSpec((3, 3 * HID, HID), lambda i: (0, 0, 0)),
            pl.BlockSpec((block, HID), lambda i: (i, 0)),
            pl.BlockSpec((block, HID), lambda i: (i, 0)),
            pl.BlockSpec((7 * HID + HID, 64), lambda i: (0, 0)),
            pl.BlockSpec((1, 64), lambda i: (0, 0)),
        ],
        out_specs=pl.BlockSpec((block, 64), lambda i: (i, 0)),
        out_shape=jax.ShapeDtypeStruct((N, 64), jnp.float32),
    )(hh2, z2, dt, hw2s, h, hh, t2_w, t2_b.reshape(1, 64))


def _tc_e_body(s_ref, w_ref, b_ref, o_ref):
    o_ref[...] = jnp.dot(s_ref[...], w_ref[...],
                         preferred_element_type=jnp.float32) + b_ref[...]


def _tc_e(s64, t3_w, t3_b):
    return pl.pallas_call(
        _tc_e_body,
        grid=(1,),
        in_specs=[
            pl.BlockSpec((4096, 64), lambda i: (0, 0)),
            pl.BlockSpec((64, 2), lambda i: (0, 0)),
            pl.BlockSpec((1, 2), lambda i: (0, 0)),
        ],
        out_specs=pl.BlockSpec((4096, 2), lambda i: (0, 0)),
        out_shape=jax.ShapeDtypeStruct((4096, 2), jnp.float32),
    )(s64, t3_w, t3_b.reshape(1, 2))


def _make_deg():
    CHC = CHT // 2

    @functools.partial(
        pl.kernel,
        out_type=jax.ShapeDtypeStruct((3 * NW * NPAD,), jnp.float32),
        mesh=_MESH,
        compiler_params=_SC_PARAMS,
        scratch_types=[
            pltpu.VMEM((2, K), jnp.int32),
            pltpu.VMEM((NPAD,), jnp.float32),
        ],
    )
    def deg_kernel(sd_hbm, out_hbm, sd1, deg_t):
        c = lax.axis_index("c")
        s = lax.axis_index("s")
        ones = jnp.full((16,), 1.0, jnp.float32)
        zeros = jnp.zeros((16,), jnp.float32)

        def rel(r, _):
            def zrow(i, _):
                deg_t[pl.ds(i * 16, 16)] = zeros
                return 0

            lax.fori_loop(0, NPAD // 16, zrow, 0)
            cbase = (r * NS + s) * CHT + c * CHC

            def chunk(i, _):
                pltpu.sync_copy(sd_hbm.at[pl.ds(2 * (cbase + i), 2)], sd1)
                for jj in range(K // 16):
                    dv = sd1[1, pl.ds(jj * 16, 16)]
                    plsc.addupdate_scatter(deg_t, [dv], ones)
                return 0

            lax.fori_loop(0, CHC, chunk, 0)
            base = ((r * NC + c) * NS + s) * NPAD
            pltpu.sync_copy(deg_t, out_hbm.at[pl.ds(base, NPAD)])
            return 0

        lax.fori_loop(0, 3, rel, 0)

    return deg_kernel


def _make_agg(FP, SUP, W=W):
    NSLOT = NC * FP
    G = CHT // SUP
    assert G % 2 == 0

    @functools.partial(
        pl.kernel,
        out_type=jax.ShapeDtypeStruct((3 * NSLOT * NPAD, W), jnp.float32),
        mesh=_MESH,
        compiler_params=_SC_PARAMS,
        scratch_types=[
            pltpu.VMEM((2 * SUP, K), jnp.int32),
            pltpu.VMEM((2 * SUP, K), jnp.int32),
            pltpu.VMEM((SUP, K), jnp.int32),
            pltpu.VMEM((SUP, K), jnp.int32),
            pltpu.VMEM((SUP * K, W), jnp.float32),
            pltpu.VMEM((SUP * K, W), jnp.float32),
            pltpu.VMEM((NPAD,), jnp.float32),
            pltpu.VMEM((NPAD,), jnp.float32),
            pltpu.VMEM((NPAD,), jnp.float32),
            pltpu.VMEM_SHARED((NPAD, W), jnp.float32),
            pltpu.SemaphoreType.DMA,
            pltpu.SemaphoreType.DMA,
            pltpu.SemaphoreType.DMA,
            pltpu.SemaphoreType.DMA,
        ],
    )
    def agg_kernel(sd_hbm, pd_hbm, ps_hbm, d_hbm, table_hbm,
                   zeros_hbm, out_hbm,
                   sd2a, sd2b, gsrca, gsrcb, rowsa, rowsb,
                   pd_t, ps_t, d_t, acc, gsem0, gsem1, osem0, osem1):
        c = lax.axis_index("c")
        s = lax.axis_index("s")
        sd2 = (sd2a, sd2b)
        gsrc = (gsrca, gsrcb)
        rows = (rowsa, rowsb)
        gsem = (gsem0, gsem1)
        osem = (osem0, osem1)

        def drain(b, sem):
            pltpu.make_async_copy(zeros_hbm.at[pl.ds(0, K)],
                                  rows[b].at[pl.ds(0, K)], sem).wait()

        def front(b, g, cbase, toff, do_drain):
            if do_drain:
                for _ in range(SUP):
                    drain(b, osem[b])
            pltpu.sync_copy(sd_hbm.at[pl.ds(2 * (cbase + g * SUP), 2 * SUP)],
                            sd2[b])
            for j in range(SUP):
                for jj in range(K // 16):
                    sl = pl.ds(jj * 16, 16)
                    gsrc[b][j, sl] = sd2[b][2 * j, sl] + toff
            for j in range(SUP):
                pltpu.async_copy(table_hbm.at[gsrc[b].at[j]],
                                 rows[b].at[pl.ds(j * K, K)], gsem[b])

        def back(b):
            for _ in range(SUP):
                drain(b, gsem[b])

            def chunk(j, _):
                for jj in range(K // 16):
                    sl = pl.ds(jj * 16, 16)
                    sv = sd2[b][2 * j, sl]
                    dv = sd2[b][2 * j + 1, sl]
                    pd = plsc.load_gather(pd_t, [dv])
                    ps = plsc.load_gather(ps_t, [sv])
                    dd = plsc.load_gather(d_t, [sv])
                    x = pd + ps
                    t = jnp.exp(-2.0 * jnp.abs(x))
                    a = (1.0 - t) / (1.0 + t)
                    f = jnp.where(x < 0.0, -a, a) * dd
                    for k16 in range(16):
                        fk = f[k16]
                        row = rows[b].at[j * K + jj * 16 + k16]
                        for q in range(W // 16):
                            sl2 = pl.ds(q * 16, 16)
                            row[sl2] = row[sl2] * fk
                pltpu.async_copy(rows[b].at[pl.ds(j * K, K)],
                                 acc.at[sd2[b].at[2 * j + 1]], osem[b],
                                 add=True)
                return 0

            lax.fori_loop(0, SUP, chunk, 0)

        def rp_body(rp, _):
            r = rp // FP
            p = rp - r * FP
            slot = c * FP + p
            toff = slot * NPAD
            pltpu.sync_copy(pd_hbm.at[pl.ds(r * NPAD, NPAD)], pd_t)
            pltpu.sync_copy(ps_hbm.at[pl.ds(r * NPAD, NPAD)], ps_t)
            pltpu.sync_copy(d_hbm.at[pl.ds(r * NPAD, NPAD)], d_t)
            pltpu.sync_copy(zeros_hbm.at[pl.ds(s * ROWS_PT, ROWS_PT)],
                            acc.at[pl.ds(s * ROWS_PT, ROWS_PT)])
            plsc.subcore_barrier()
            cbase = (r * NS + s) * CHT
            front(0, 0, cbase, toff, False)
            front(1, 1, cbase, toff, False)

            def pair(i, _):
                back(0)
                back(1)
                g2 = 2 * i + 2

                @pl.when(g2 < G)
                def _():
                    front(0, g2, cbase, toff, True)
                    front(1, g2 + 1, cbase, toff, True)

                return 0

            lax.fori_loop(0, G // 2, pair, 0)
            for b in range(2):
                for _ in range(SUP):
                    drain(b, osem[b])
            plsc.subcore_barrier()
            base = (r * NSLOT + slot) * NPAD + s * ROWS_PT
            pltpu.sync_copy(acc.at[pl.ds(s * ROWS_PT, ROWS_PT)],
                            out_hbm.at[pl.ds(base, ROWS_PT)])
            plsc.subcore_barrier()
            return 0

        lax.fori_loop(0, 3 * FP, rp_body, 0)

    return agg_kernel


def _make_gather():
    @functools.partial(
        pl.kernel,
        out_type=jax.ShapeDtypeStruct((4096, 64), jnp.float32),
        mesh=_MESH,
        compiler_params=_SC_PARAMS,
        scratch_types=[
            pltpu.VMEM((K,), jnp.int32),
            pltpu.VMEM((K, 64), jnp.float32),
            pltpu.SemaphoreType.DMA,
        ],
    )
    def gather_kernel(y_hbm, nodes_hbm, out_hbm, idx, rows, sem):
        c = lax.axis_index("c")
        s = lax.axis_index("s")
        w = s * NC + c
        base = w * K
        pltpu.sync_copy(nodes_hbm.at[pl.ds(base, K)], idx)
        pltpu.async_copy(y_hbm.at[idx], rows, sem).wait()
        pltpu.sync_copy(rows, out_hbm.at[pl.ds(base, K)])

    return gather_kernel


_DEG = _make_deg()
_AGG1 = _make_agg(1, 2, 64)
_AGG2 = _make_agg(3, 2, 64)
_GATHER = _make_gather()


def _pad_edges(ei):
    pad = jnp.full((EPAD - E,), DUMMY, jnp.int32)
    src = jnp.concatenate([ei[0], pad]).reshape(-1, K)
    dst = jnp.concatenate([ei[1], pad]).reshape(-1, K)
    return jnp.stack([src, dst], axis=1).reshape(-1, K)


def _pad_rows(x):
    return jnp.pad(x, ((0, NPAD - N), (0, 0)))


def _gate_mats(ws, bs, dim):
    cols = []
    for w_, _ in zip(ws, bs):
        cols.append(w_[:dim, 0])
        cols.append(w_[dim:, 0])
    cols.append(jnp.zeros((dim,), jnp.float32))
    cols.append(jnp.zeros((dim,), jnp.float32))
    g = jnp.stack(cols, axis=1)
    gb = jnp.stack([bs[0][0], jnp.float32(0), bs[1][0], jnp.float32(0),
                    bs[2][0], jnp.float32(0), jnp.float32(0),
                    jnp.float32(0)])
    return g, gb


def _split_pdps(pp):
    pd = jnp.pad(jnp.stack([pp[:, 0], pp[:, 2], pp[:, 4]]),
                 ((0, 0), (0, NPAD - N))).reshape(-1)
    ps = jnp.pad(jnp.stack([pp[:, 1], pp[:, 3], pp[:, 5]]),
                 ((0, 0), (0, NPAD - N))).reshape(-1)
    return pd, ps


def _slices(x, w, nslices):
    return jnp.concatenate([x[:, w * q:w * (q + 1)] for q in range(nslices)],
                           axis=0)


def kernel(nodes, h, edge_index1, edge_index2, edge_index3, t1_w, t1_b,
           gate1_w1, gate1_b1, gate1_w2, gate1_b2, gate1_w3, gate1_b3,
           hw1_1, hw1_2, hw1_3,
           gate2_w1, gate2_b1, gate2_w2, gate2_b2, gate2_w3, gate2_b3,
           hw2_1, hw2_2, hw2_3, t2_w, t2_b, t3_w, t3_b):
    sd3 = jnp.concatenate([_pad_edges(edge_index1),
                           _pad_edges(edge_index2),
                           _pad_edges(edge_index3)])

    z64 = jnp.zeros((NPAD, 64), jnp.float32)

    degp = _DEG(sd3).reshape(3, NW, NPAD)
    d3 = _tc_d(degp)
    dt = d3[:, :N].T
    d3f = d3.reshape(-1)

    g1, gb1 = _gate_mats([gate1_w1, gate1_w2, gate1_w3],
                         [gate1_b1, gate1_b2, gate1_b3], HID)
    hh, pp1 = _tc_a(h, t1_w, t1_b, g1, gb1)
    pd1, ps1 = _split_pdps(pp1)

    table1 = _slices(_pad_rows(hh), 64, 2)
    zr1 = _AGG1(sd3, pd1, ps1, d3f, table1, z64)
    z1 = (zr1.reshape(3, 2, NPAD, 64).transpose(0, 2, 1, 3)
          .reshape(3, NPAD, HID)[:, :N, :])

    g2, gb2 = _gate_mats([gate2_w1, gate2_w2, gate2_w3],
                         [gate2_b1, gate2_b2, gate2_b3], 3 * HID)
    hw1s = jnp.stack([hw1_1, hw1_2, hw1_3])
    hh2, pp2 = _tc_b(hh, z1, dt, hw1s, g2, gb2)
    pd2, ps2 = _split_pdps(pp2)

    table2 = _slices(_pad_rows(hh2), 64, 6)
    zr2 = _AGG2(sd3, pd2, ps2, d3f, table2, z64)
    z2 = (zr2.reshape(3, 6, NPAD, 64).transpose(0, 2, 1, 3)
          .reshape(3, NPAD, 3 * HID)[:, :N, :])

    hw2s = jnp.stack([hw2_1, hw2_2, hw2_3])
    y = _tc_f(hh2, z2, dt, hw2s, h, hh, t2_w, t2_b)

    s64 = _GATHER(y, nodes)
    scores = _tc_e(s64, t3_w, t3_b)
    return (scores, s64)
